# baseline (speedup 1.0000x reference)
import functools

import jax
import jax.numpy as jnp
from jax import lax
from jax.experimental import pallas as pl
from jax.experimental.pallas import tpu as pltpu
from jax.experimental.pallas import tpu_sc as plsc

D = 64
LANES = 16
CHUNK = 128


def _make_body(n_flat, seq, n_cores, n_subcores):
  n_workers = n_cores * n_subcores
  per_w = n_flat // n_workers
  n_it = per_w // CHUNK
  assert per_w % CHUNK == 0 and per_w % seq == 0 and n_it % 2 == 0
  idx_rows_w = per_w // CHUNK
  pe_rows = (seq // 2 + 7) // 8 * 8

  mesh = plsc.VectorSubcoreMesh(core_axis_name="c", subcore_axis_name="s")

  @functools.partial(
      pl.kernel,
      out_type=jax.ShapeDtypeStruct((n_flat // 2, 2 * D), jnp.float32),
      mesh=mesh,
      compiler_params=pltpu.CompilerParams(use_tc_tiling_on_sc=True),
      scratch_types=[
          pltpu.VMEM((idx_rows_w, CHUNK), jnp.int32),
          pltpu.VMEM((CHUNK,), jnp.int32),
          pltpu.VMEM((CHUNK,), jnp.int32),
          pltpu.VMEM((CHUNK, 2 * D), jnp.float32),
          pltpu.VMEM((CHUNK, 2 * D), jnp.float32),
          pltpu.VMEM((CHUNK // 2, 2 * D), jnp.float32),
          pltpu.VMEM((CHUNK // 2, 2 * D), jnp.float32),
          pltpu.VMEM((pe_rows, 2 * D), jnp.float32),
          pltpu.SemaphoreType.DMA,
          pltpu.SemaphoreType.DMA,
          pltpu.SemaphoreType.DMA,
          pltpu.SemaphoreType.DMA,
      ],
  )
  def body(idx_hbm, table_hbm, pos_hbm, out_hbm,
           idx_v, idxh0, idxh1, gath0, gath1, wr0, wr1, pe_v,
           sem_g0, sem_g1, sem_w0, sem_w1):
    idxh = (idxh0, idxh1)
    gath = (gath0, gath1)
    wr = (wr0, wr1)
    sem_g = (sem_g0, sem_g1)
    sem_w = (sem_w0, sem_w1)

    wid = lax.axis_index("s") * n_cores + lax.axis_index("c")
    row0 = wid * idx_rows_w
    out0 = wid * (per_w // 2)

    pltpu.sync_copy(idx_hbm.at[pl.ds(row0, idx_rows_w)], idx_v)
    pltpu.sync_copy(pos_hbm.at[pl.ds(0, pe_rows)], pe_v)

    def fire_gathers(t, b):
      for q in range(CHUNK // LANES):
        sl = pl.ds(q * LANES, LANES)
        idxh[b][sl] = idx_v[t, sl] >> 1
      pltpu.async_copy(table_hbm.at[idxh[b]], gath[b], sem_g[b])

    def drain_gathers(b):
      pltpu.make_async_copy(table_hbm.at[idxh[b]], gath[b], sem_g[b]).wait()

    def fire_write(t, b):
      pltpu.async_copy(wr[b], out_hbm.at[pl.ds(out0 + t * (CHUNK // 2),
                                               CHUNK // 2)], sem_w[b])

    def drain_write(t, b):
      pltpu.make_async_copy(wr[b], out_hbm.at[pl.ds(out0 + t * (CHUNK // 2),
                                                    CHUNK // 2)],
                            sem_w[b]).wait()

    def select_add_pack(t, b):
      phase = lax.rem(t * CHUNK, seq)

      def grp_body(g, carry):
        r0 = g * LANES
        tid_vec = idx_v[t, pl.ds(r0, LANES)]
        for l in range(LANES):
          r = r0 + l
          src0 = (tid_vec[l] & 1) * D
          s_raw = phase + r
          s = jnp.where(s_raw < seq, s_raw, s_raw - seq)
          pcol = (s & 1) * D
          prow = s >> 1
          dcol = (r & 1) * D
          drow = r >> 1
          for q in range(D // LANES):
            v = (gath[b][r, pl.ds(src0 + q * LANES, LANES)]
                 + pe_v[prow, pl.ds(pcol + q * LANES, LANES)])
            wr[b][drow, pl.ds(dcol + q * LANES, LANES)] = v
        return carry
      lax.fori_loop(0, CHUNK // LANES, grp_body, None, unroll=False)

    fire_gathers(0, 0)

    def step(t2, carry):
      t = 2 * t2

      @pl.when(t2 > 0)
      def _():
        drain_write(t - 1, 1)
      fire_gathers(t + 1, 1)
      drain_gathers(0)
      select_add_pack(t, 0)
      fire_write(t, 0)

      t = 2 * t2 + 1
      drain_write(t - 1, 0)

      @pl.when(t2 < n_it // 2 - 1)
      def _():
        fire_gathers(t + 1, 0)
      drain_gathers(1)
      select_add_pack(t, 1)
      fire_write(t, 1)
      return carry

    lax.fori_loop(0, n_it // 2, step, None, unroll=False)
    drain_write(n_it - 1, 1)

  return body


def kernel(inputs, table, pos_encoding):
  b, s = inputs.shape
  n_flat = b * s
  v, d = table.shape
  pm, _ = pos_encoding.shape
  info = plsc.get_sparse_core_info()
  idx2d = inputs.reshape(n_flat // CHUNK, CHUNK).astype(jnp.int32)
  t2 = table.reshape(v // 2, 2 * d)
  pos2 = pos_encoding.reshape(pm // 2, 2 * d)
  body = _make_body(n_flat, s, info.num_cores, info.num_subcores)
  out = body(idx2d, t2, pos2)
  return out.reshape(b, s, d)


# speedup vs baseline: 1.2316x; 1.2316x over previous
import functools

import jax
import jax.numpy as jnp
from jax import lax
from jax.experimental import pallas as pl
from jax.experimental.pallas import tpu as pltpu
from jax.experimental.pallas import tpu_sc as plsc

LANES = 16
CHUNK = 128


def _make_body(n_flat, seq, d, n_cores, n_subcores):
  n_workers = n_cores * n_subcores
  per_w = n_flat // n_workers
  n_it = per_w // CHUNK
  assert n_flat % n_workers == 0 and per_w % CHUNK == 0
  assert per_w % seq == 0 and n_it % 4 == 0
  assert seq % 8 == 0 and d % LANES == 0
  idx_rows_w = per_w // CHUNK
  pe_rows = seq + CHUNK

  mesh = plsc.VectorSubcoreMesh(core_axis_name="c", subcore_axis_name="s")

  @functools.partial(
      pl.kernel,
      out_type=jax.ShapeDtypeStruct((n_flat, d), jnp.float32),
      mesh=mesh,
      compiler_params=pltpu.CompilerParams(use_tc_tiling_on_sc=True),
      scratch_types=[
          pltpu.VMEM(((idx_rows_w // 2 + 7) // 8 * 8, CHUNK), jnp.int32),
          pltpu.VMEM((CHUNK,), jnp.int32),
          pltpu.VMEM((CHUNK,), jnp.int32),
          pltpu.VMEM((CHUNK, 2 * d), jnp.float32),
          pltpu.VMEM((CHUNK, 2 * d), jnp.float32),
          pltpu.VMEM((CHUNK, d), jnp.float32),
          pltpu.VMEM((CHUNK, d), jnp.float32),
          pltpu.VMEM((pe_rows, d), jnp.float32),
          pltpu.SemaphoreType.DMA,
          pltpu.SemaphoreType.DMA,
          pltpu.SemaphoreType.DMA,
          pltpu.SemaphoreType.DMA,
      ],
  )
  def body(idx_hbm, table_hbm, pos_hbm, out_hbm,
           idx_v, ib0, ib1, g0, g1, w0, w1, pe_v,
           sem_g0, sem_g1, sem_w0, sem_w1):
    idxb = (ib0, ib1)
    gath = (g0, g1)
    wr = (w0, w1)
    sem_g = (sem_g0, sem_g1)
    sem_w = (sem_w0, sem_w1)

    wid = lax.axis_index("s") * n_cores + lax.axis_index("c")
    row0 = wid * idx_rows_w
    out0 = wid * per_w
    split = (idx_rows_w // 2 + 7) // 8 * 8

    pltpu.sync_copy(idx_hbm.at[pl.ds(row0, split)], idx_v)
    pltpu.sync_copy(pos_hbm.at[pl.ds(0, seq)], pe_v.at[pl.ds(0, seq)])
    pltpu.sync_copy(pos_hbm.at[pl.ds(0, CHUNK)], pe_v.at[pl.ds(seq, CHUNK)])

    def fire_gather(t, k):
      tr = jnp.where(t < split, t, t - split)
      for q in range(CHUNK // LANES):
        sl = pl.ds(q * LANES, LANES)
        idxb[k][sl] = idx_v[tr, sl]
      pltpu.async_copy(table_hbm.at[idxb[k]], gath[k], sem_g[k])

    def drain_gather(k):
      pltpu.make_async_copy(table_hbm.at[idxb[k]], gath[k], sem_g[k]).wait()

    def fire_write(t, k):
      pltpu.async_copy(wr[k], out_hbm.at[pl.ds(out0 + t * CHUNK, CHUNK)],
                       sem_w[k])

    def drain_write(t, k):
      pltpu.make_async_copy(wr[k], out_hbm.at[pl.ds(out0 + t * CHUNK, CHUNK)],
                            sem_w[k]).wait()

    def add_pe(t, k):
      phase = lax.rem(t * CHUNK, seq)

      def row(r, carry):
        for q in range(d // LANES):
          sl = pl.ds(q * LANES, LANES)
          wr[k][r, sl] = gath[k][r, sl] + pe_v[phase + r, sl]
        return carry
      lax.fori_loop(0, CHUNK, row, None, unroll=False)

    fire_gather(0, 0)

    def step(t2, carry):
      t = 2 * t2
      @pl.when(t2 > 0)
      def _():
        drain_write(t - 1, 1)
      fire_gather(t + 1, 1)
      drain_gather(0)
      add_pe(t, 0)
      fire_write(t, 0)

      drain_write(t, 0)

      @pl.when(2 * t2 + 2 == split)
      def _():
        pltpu.sync_copy(idx_hbm.at[pl.ds(row0 + split, idx_rows_w - split)],
                        idx_v.at[pl.ds(0, idx_rows_w - split)])

      @pl.when(t2 < n_it // 2 - 1)
      def _():
        fire_gather(t + 2, 0)
      drain_gather(1)
      add_pe(t + 1, 1)
      fire_write(t + 1, 1)
      return carry

    lax.fori_loop(0, n_it // 2, step, None, unroll=False)
    drain_write(n_it - 1, 1)

  return body


def kernel(inputs, table, pos_encoding):
  b, s = inputs.shape
  n_flat = b * s
  v, d = table.shape
  info = plsc.get_sparse_core_info()
  idx2d = inputs.reshape(n_flat // CHUNK, CHUNK).astype(jnp.int32)
  table128 = jnp.pad(table, ((0, 0), (0, d)))
  body = _make_body(n_flat, s, d, info.num_cores, info.num_subcores)
  out = body(idx2d, table128, pos_encoding)
  return out.reshape(b, s, d)
